# baseline (speedup 1.0000x reference)
import jax
import jax.numpy as jnp
from jax import lax
from jax.experimental import pallas as pl
from jax.experimental.pallas import tpu as pltpu
from jax.experimental.pallas import tpu_sc as plsc

NUM_TOKENS = 2048
DIM = 768
NUM_LVL = 63
NC, NS, L = 2, 16, 16
NW = NC * NS
TPW = NUM_TOKENS // NW
CH = DIM // L


NCHUNK = 4
TPC = TPW // NCHUNK


def _sc_body(z_hbm, zq_hbm, idx_hbm, buf, idxbuf, sin, szq, sidx):
    wid = lax.axis_index("s") * NC + lax.axis_index("c")
    base = wid * TPW
    ins = [pltpu.async_copy(z_hbm.at[pl.ds(base + g * TPC, TPC)],
                            buf.at[pl.ds(g * TPC, TPC)], sin.at[g])
           for g in range(NCHUNK)]
    outs = []

    def tok_body(r, carry):
        vs = [buf[r, pl.ds(L * j, L)] for j in range(CH)]
        acc_s = vs[0]
        acc_q = vs[0] * vs[0]
        for j in range(1, CH):
            acc_s = acc_s + vs[j]
            acc_q = acc_q + vs[j] * vs[j]
        sv = jnp.broadcast_to(jnp.sum(acc_s), (L,))
        qv = jnp.broadcast_to(jnp.sum(acc_q), (L,))
        mean = sv * (1.0 / DIM)
        var = jnp.maximum((qv - sv * mean) * (1.0 / (DIM - 1)), 0.0)
        bits = lax.bitcast_convert_type(var, jnp.int32)
        y = lax.bitcast_convert_type(
            jnp.int32(0x5F3759DF) - (bits >> 1), jnp.float32)
        for _ in range(2):
            y = y * (1.5 - 0.5 * var * y * y)
        std = var * y + 1e-5
        a = (NUM_LVL - 1.0) / 2.0 / std
        b1 = ((NUM_LVL - 1.0) / 2.0 + 0.5) - mean * a
        scale = std * (2.0 / (NUM_LVL - 1.0))
        shift = mean - std
        for j in range(CH):
            tq = jnp.minimum(jnp.maximum(vs[j] * a + b1, 0.0),
                             float(NUM_LVL - 1))
            ti = tq.astype(jnp.int32)
            idxbuf[r, pl.ds(L * j, L)] = ti
            buf[r, pl.ds(L * j, L)] = ti.astype(jnp.float32) * scale + shift
        return carry

    for g in range(NCHUNK):
        ins[g].wait()
        lax.fori_loop(g * TPC, (g + 1) * TPC, tok_body, 0)
        outs.append(pltpu.async_copy(
            buf.at[pl.ds(g * TPC, TPC)],
            zq_hbm.at[pl.ds(base + g * TPC, TPC)], szq.at[g]))
        outs.append(pltpu.async_copy(
            idxbuf.at[pl.ds(g * TPC, TPC)],
            idx_hbm.at[pl.ds(base + g * TPC, TPC)], sidx.at[g]))
    for d in outs:
        d.wait()


def kernel(z):
    zf = z.reshape(NUM_TOKENS, DIM)
    mesh = plsc.VectorSubcoreMesh(
        core_axis_name="c", subcore_axis_name="s",
        num_cores=NC, num_subcores=NS)
    fn = pl.kernel(
        _sc_body,
        out_type=(
            jax.ShapeDtypeStruct((NUM_TOKENS, DIM), jnp.float32),
            jax.ShapeDtypeStruct((NUM_TOKENS, DIM), jnp.int32),
        ),
        mesh=mesh,
        compiler_params=pltpu.CompilerParams(needs_layout_passes=False),
        scratch_types=[
            pltpu.VMEM((TPW, DIM), jnp.float32),
            pltpu.VMEM((TPW, DIM), jnp.int32),
            pltpu.SemaphoreType.DMA((NCHUNK,)),
            pltpu.SemaphoreType.DMA((NCHUNK,)),
            pltpu.SemaphoreType.DMA((NCHUNK,)),
        ],
    )
    zq, idx = fn(zf)
    return zq.reshape(z.shape), idx.reshape(z.shape)


# speedup vs baseline: 1.0700x; 1.0700x over previous
import jax
import jax.numpy as jnp
from jax import lax
from jax.experimental import pallas as pl
from jax.experimental.pallas import tpu as pltpu
from jax.experimental.pallas import tpu_sc as plsc

NUM_TOKENS = 2048
DIM = 768
NUM_LVL = 63
NC, NS, L = 2, 16, 16
NW = NC * NS
TPW = NUM_TOKENS // NW
CH = DIM // L


NCHUNK = 4
TPC = TPW // NCHUNK


def _sc_body(z_hbm, zq_hbm, idx_hbm, buf, idxbuf, sin, szq, sidx):
    wid = lax.axis_index("s") * NC + lax.axis_index("c")
    base = wid * TPW
    ins = [pltpu.async_copy(z_hbm.at[pl.ds(base + g * TPC, TPC)],
                            buf.at[pl.ds(g * TPC, TPC)], sin.at[g])
           for g in range(NCHUNK)]
    outs = []

    def tok_body(r):
        acc_s = jnp.zeros((L,), jnp.float32)
        acc_q = jnp.zeros((L,), jnp.float32)
        for j in range(CH):
            v = buf[r, pl.ds(L * j, L)]
            acc_s = acc_s + v
            acc_q = acc_q + v * v
        sv = jnp.broadcast_to(jnp.sum(acc_s), (L,))
        qv = jnp.broadcast_to(jnp.sum(acc_q), (L,))
        mean = sv * (1.0 / DIM)
        var = jnp.maximum((qv - sv * mean) * (1.0 / (DIM - 1)), 0.0)
        bits = lax.bitcast_convert_type(var, jnp.int32)
        y = lax.bitcast_convert_type(
            jnp.int32(0x5F3759DF) - (bits >> 1), jnp.float32)
        for _ in range(2):
            y = y * (1.5 - 0.5 * var * y * y)
        std = var * y + 1e-5
        a = (NUM_LVL - 1.0) / 2.0 / std
        b1 = ((NUM_LVL - 1.0) / 2.0 + 0.5) - mean * a
        scale = std * (2.0 / (NUM_LVL - 1.0))
        shift = mean - std
        for j in range(CH):
            v = buf[r, pl.ds(L * j, L)]
            tq = jnp.minimum(jnp.maximum(v * a + b1, 0.0),
                             float(NUM_LVL - 1))
            ti = tq.astype(jnp.int32)
            idxbuf[r, pl.ds(L * j, L)] = ti
            buf[r, pl.ds(L * j, L)] = ti.astype(jnp.float32) * scale + shift

    for g in range(NCHUNK):
        ins[g].wait()
        plsc.parallel_loop(g * TPC, (g + 1) * TPC, 1, unroll=2)(tok_body)
        outs.append(pltpu.async_copy(
            buf.at[pl.ds(g * TPC, TPC)],
            zq_hbm.at[pl.ds(base + g * TPC, TPC)], szq.at[g]))
        outs.append(pltpu.async_copy(
            idxbuf.at[pl.ds(g * TPC, TPC)],
            idx_hbm.at[pl.ds(base + g * TPC, TPC)], sidx.at[g]))
    for d in outs:
        d.wait()


def kernel(z):
    zf = z.reshape(NUM_TOKENS, DIM)
    mesh = plsc.VectorSubcoreMesh(
        core_axis_name="c", subcore_axis_name="s",
        num_cores=NC, num_subcores=NS)
    fn = pl.kernel(
        _sc_body,
        out_type=(
            jax.ShapeDtypeStruct((NUM_TOKENS, DIM), jnp.float32),
            jax.ShapeDtypeStruct((NUM_TOKENS, DIM), jnp.int32),
        ),
        mesh=mesh,
        compiler_params=pltpu.CompilerParams(needs_layout_passes=False),
        scratch_types=[
            pltpu.VMEM((TPW, DIM), jnp.float32),
            pltpu.VMEM((TPW, DIM), jnp.int32),
            pltpu.SemaphoreType.DMA((NCHUNK,)),
            pltpu.SemaphoreType.DMA((NCHUNK,)),
            pltpu.SemaphoreType.DMA((NCHUNK,)),
        ],
    )
    zq, idx = fn(zf)
    return zq.reshape(z.shape), idx.reshape(z.shape)


# speedup vs baseline: 1.0732x; 1.0030x over previous
import jax
import jax.numpy as jnp
from jax import lax
from jax.experimental import pallas as pl
from jax.experimental.pallas import tpu as pltpu
from jax.experimental.pallas import tpu_sc as plsc

NUM_TOKENS = 2048
DIM = 768
NUM_LVL = 63
NC, NS, L = 2, 16, 16
NW = NC * NS
TPW = NUM_TOKENS // NW
CH = DIM // L


NCHUNK = 2
TPC = TPW // NCHUNK


def _sc_body(z_hbm, zq_hbm, idx_hbm, buf, idxbuf, sin, szq, sidx):
    wid = lax.axis_index("s") * NC + lax.axis_index("c")
    base = wid * TPW
    ins = [pltpu.async_copy(z_hbm.at[pl.ds(base + g * TPC, TPC)],
                            buf.at[pl.ds(g * TPC, TPC)], sin.at[g])
           for g in range(NCHUNK)]
    outs = []

    def tok_body(r):
        acc_s = jnp.zeros((L,), jnp.float32)
        acc_q = jnp.zeros((L,), jnp.float32)
        for j in range(CH):
            v = buf[r, pl.ds(L * j, L)]
            acc_s = acc_s + v
            acc_q = acc_q + v * v
        sv = jnp.broadcast_to(jnp.sum(acc_s), (L,))
        qv = jnp.broadcast_to(jnp.sum(acc_q), (L,))
        mean = sv * (1.0 / DIM)
        var = jnp.maximum((qv - sv * mean) * (1.0 / (DIM - 1)), 0.0)
        bits = lax.bitcast_convert_type(var, jnp.int32)
        y = lax.bitcast_convert_type(
            jnp.int32(0x5F3759DF) - (bits >> 1), jnp.float32)
        for _ in range(2):
            y = y * (1.5 - 0.5 * var * y * y)
        std = var * y + 1e-5
        a = (NUM_LVL - 1.0) / 2.0 / std
        b1 = ((NUM_LVL - 1.0) / 2.0 + 0.5) - mean * a
        scale = std * (2.0 / (NUM_LVL - 1.0))
        shift = mean - std
        for j in range(CH):
            v = buf[r, pl.ds(L * j, L)]
            tq = jnp.minimum(jnp.maximum(v * a + b1, 0.0),
                             float(NUM_LVL - 1))
            ti = tq.astype(jnp.int32)
            idxbuf[r, pl.ds(L * j, L)] = ti
            buf[r, pl.ds(L * j, L)] = ti.astype(jnp.float32) * scale + shift

    for g in range(NCHUNK):
        ins[g].wait()
        plsc.parallel_loop(g * TPC, (g + 1) * TPC, 1, unroll=4)(tok_body)
        outs.append(pltpu.async_copy(
            buf.at[pl.ds(g * TPC, TPC)],
            zq_hbm.at[pl.ds(base + g * TPC, TPC)], szq.at[g]))
        outs.append(pltpu.async_copy(
            idxbuf.at[pl.ds(g * TPC, TPC)],
            idx_hbm.at[pl.ds(base + g * TPC, TPC)], sidx.at[g]))
    for d in outs:
        d.wait()


def kernel(z):
    zf = z.reshape(NUM_TOKENS, DIM)
    mesh = plsc.VectorSubcoreMesh(
        core_axis_name="c", subcore_axis_name="s",
        num_cores=NC, num_subcores=NS)
    fn = pl.kernel(
        _sc_body,
        out_type=(
            jax.ShapeDtypeStruct((NUM_TOKENS, DIM), jnp.float32),
            jax.ShapeDtypeStruct((NUM_TOKENS, DIM), jnp.int32),
        ),
        mesh=mesh,
        compiler_params=pltpu.CompilerParams(needs_layout_passes=False),
        scratch_types=[
            pltpu.VMEM((TPW, DIM), jnp.float32),
            pltpu.VMEM((TPW, DIM), jnp.int32),
            pltpu.SemaphoreType.DMA((NCHUNK,)),
            pltpu.SemaphoreType.DMA((NCHUNK,)),
            pltpu.SemaphoreType.DMA((NCHUNK,)),
        ],
    )
    zq, idx = fn(zf)
    return zq.reshape(z.shape), idx.reshape(z.shape)
